# baseline (speedup 1.0000x reference)
import functools

import jax
import jax.numpy as jnp
from jax import lax
from jax.experimental import pallas as pl
from jax.experimental.pallas import tpu as pltpu
from jax.experimental.pallas import tpu_sc as plsc

B = 4
D = 96
NQ = 25
BN_C = float(1.0 / (1.0 + 1e-5) ** 0.5)


def _dot(a, b):
    return jnp.dot(a, b, preferred_element_type=jnp.float32)


def _mlp_attention(Xc, nv, w1, w2, w3, w4, w5, b5):
    np_, _ = Xc.shape
    lrelu = lambda v: jax.nn.leaky_relu(v, 0.01)
    G = 4 if np_ % 32 == 0 else 2
    g = np_ // G

    def chain(p):
        h = lrelu(BN_C * _dot(p, w1))
        h = lrelu(BN_C * _dot(h, w2))
        h = lrelu(BN_C * _dot(h, w3))
        h = lrelu(BN_C * _dot(h, w4))
        return _dot(h, w5) + b5

    pairs = [(ci, cj) for ci in range(G) for cj in range(ci, G)]
    difs = []
    for ci, cj in pairs:
        xi = Xc[ci * g:(ci + 1) * g]
        xj = Xc[cj * g:(cj + 1) * g]
        dif = jnp.abs(xi[:, None, :] - xj[None, :, :])
        difs.append(dif.reshape(g * g, D))
    LG = chain(jnp.concatenate(difs, axis=0))
    blocks = {}
    for k, (ci, cj) in enumerate(pairs):
        blocks[(ci, cj)] = LG[k * g * g:(k + 1) * g * g].reshape(g, g)
    rows = []
    for ci in range(G):
        row = [blocks[(ci, cj)] if ci <= cj else blocks[(cj, ci)].T
               for cj in range(G)]
        rows.append(jnp.concatenate(row, axis=1))
    logits = jnp.concatenate(rows, axis=0)
    if nv < np_:
        jj = jax.lax.broadcasted_iota(jnp.int32, (np_, np_), 1)
        logits = jnp.where(jj < nv, logits, -1e30)
    return jax.nn.softmax(logits, axis=-1)


def _stage(Xc, nv, w1, w2, w3, w4, w5, b5, gw, gb):
    A = _mlp_attention(Xc, nv, w1, w2, w3, w4, w5, b5)
    return _dot(_dot(A, Xc), gw) + gb


def _score_row(Xc, wp, bp):
    return lax.dot_general(
        wp, Xc, (((0,), (1,)), ((), ())),
        preferred_element_type=jnp.float32) + bp


def _tc_a_kernel(x_ref, *refs):
    ws = [r[:] for r in refs[:18]]
    xc_out, z_out = refs[18], refs[19]
    Xc = x_ref[0]
    Xc = _stage(Xc, 128, *ws[0:8])
    Xc = _stage(Xc, 128, *ws[8:16])
    xc_out[0] = jnp.concatenate(
        [Xc, jnp.zeros((Xc.shape[0], 128 - D), jnp.float32)], axis=1)
    z_out[0] = _score_row(Xc, ws[16], ws[17])


def _tc_mid_kernel(np_use, nv, with_score, rows_ref, ssel_ref, *refs):
    nw = 10 if with_score else 8
    ws = [r[:] for r in refs[:nw]]
    Xc = (rows_ref[0] * ssel_ref[0])[:np_use, :D]
    Xc = _stage(Xc, nv, *ws[:8])
    if with_score:
        refs[nw][0] = jnp.concatenate(
            [Xc, jnp.zeros((np_use, 128 - D), jnp.float32)], axis=1)
        z = _score_row(Xc, ws[8], ws[9])
        if np_use < 128:
            z = jnp.concatenate(
                [z, jnp.zeros((1, 128 - np_use), jnp.float32)], axis=1)
        refs[nw + 1][0] = z
    else:
        refs[nw][0] = Xc


def _mlp_args(params, name):
    mp = params[name]
    return [mp['w1'], mp['w2'], mp['w3'], mp['w4'], mp['w5'],
            mp['b5'].reshape(1, 1)]


def _gcn_args(params, name):
    gp = params[name]
    return [gp['w1'], gp['b1'].reshape(1, D)]


def _pool_args(params, name):
    pp = params[name]
    return [pp['wp'], pp['bp'].reshape(1, 1)]


def _wspecs(args):
    return [pl.BlockSpec(a.shape, lambda b, _n=a.ndim: (0,) * _n)
            for a in args]


def _tc_a(X, params):
    args = (_mlp_args(params, 'start_mlp') + _gcn_args(params, 'start_gcn')
            + _mlp_args(params, 'down_mlp_0') + _gcn_args(params, 'down_gcn_0')
            + _pool_args(params, 'pool_0'))
    return pl.pallas_call(
        _tc_a_kernel,
        grid=(B,),
        in_specs=[pl.BlockSpec((1, 128, D), lambda b: (b, 0, 0))]
        + _wspecs(args),
        out_specs=[pl.BlockSpec((1, 128, 128), lambda b: (b, 0, 0)),
                   pl.BlockSpec((1, 1, 128), lambda b: (b, 0, 0))],
        out_shape=[jax.ShapeDtypeStruct((B, 128, 128), jnp.float32),
                   jax.ShapeDtypeStruct((B, 1, 128), jnp.float32)],
    )(X, *args)


def _tc_mid(rows, ssel, params, npad, np_use, nv, mlp, gcn, pool):
    args = _mlp_args(params, mlp) + _gcn_args(params, gcn)
    if pool is not None:
        args += _pool_args(params, pool)
    oc = 128 if pool is not None else D
    out_specs = [pl.BlockSpec((1, np_use, oc), lambda b: (b, 0, 0))]
    out_shape = [jax.ShapeDtypeStruct((B, np_use, oc), jnp.float32)]
    if pool is not None:
        out_specs.append(pl.BlockSpec((1, 1, 128), lambda b: (b, 0, 0)))
        out_shape.append(jax.ShapeDtypeStruct((B, 1, 128), jnp.float32))
    res = pl.pallas_call(
        functools.partial(_tc_mid_kernel, np_use, nv, pool is not None),
        grid=(B,),
        in_specs=[pl.BlockSpec((1, npad, 128), lambda b: (b, 0, 0)),
                  pl.BlockSpec((1, npad, 1), lambda b: (b, 0, 0))]
        + _wspecs(args),
        out_specs=out_specs,
        out_shape=out_shape,
    )(rows, ssel.reshape(B, npad, 1), *args)
    return res if pool is not None else res[0]


def _sc_pool_call(x_flat, z, np_in, nv, np_out):
    ns = nv - NQ
    kk = int(0.7 * ns)
    nnew = kk + NQ
    niv = (ns + 15) // 16

    mesh = plsc.VectorSubcoreMesh(core_axis_name="c", subcore_axis_name="s",
                                  num_cores=2, num_subcores=16)

    @functools.partial(
        pl.kernel,
        out_type=(jax.ShapeDtypeStruct((B * 128, 128), jnp.float32),
                  jax.ShapeDtypeStruct((B, 128), jnp.float32)),
        mesh=mesh,
        compiler_params=pltpu.CompilerParams(needs_layout_passes=False),
        scratch_types=[
            pltpu.VMEM((128,), jnp.float32),
            pltpu.VMEM((128,), jnp.int32),
            pltpu.VMEM((128,), jnp.float32),
            pltpu.VMEM((128, 128), jnp.float32),
            pltpu.SemaphoreType.DMA,
        ],
    )
    def sc_pool(x_hbm, z_hbm, rows_hbm, ssel_hbm, s_v, idx_v, ssel_v,
                rows_v, sem):
        cid = lax.axis_index("c")
        sid = lax.axis_index("s")
        wid = sid * 2 + cid

        @pl.when(wid < B)
        def _body():
            b = wid
            base = b * np_in
            pltpu.sync_copy(z_hbm.at[b], s_v)
            iota = lax.iota(jnp.int32, 16)
            for v in range(8):
                zz = s_v[pl.ds(16 * v, 16)]
                s_v[pl.ds(16 * v, 16)] = 1.0 / (1.0 + jnp.exp(-zz * 0.01))
            for v in range(8):
                idx_v[pl.ds(16 * v, 16)] = jnp.zeros((16,), jnp.int32) + base
                ssel_v[pl.ds(16 * v, 16)] = jnp.zeros((16,), jnp.float32)
            sis = [s_v[pl.ds(16 * v, 16)] for v in range(niv)]
            iis = [iota + 16 * v for v in range(niv)]

            def jbody(j, ranks):
                sj = plsc.load_gather(s_v, [jnp.zeros((16,), jnp.int32) + j])
                out = []
                for v in range(niv):
                    cond = ((sj > sis[v])
                            | ((sj == sis[v]) & (j < iis[v])))
                    out.append(ranks[v] + cond.astype(jnp.int32))
                return tuple(out)

            ranks = lax.fori_loop(
                0, ns, jbody,
                tuple(jnp.zeros((16,), jnp.int32) for _ in range(niv)))
            for v in range(niv):
                m = (ranks[v] < kk) & (iis[v] < ns)
                plsc.store_scatter(idx_v, [ranks[v]], iis[v] + base, mask=m)
                plsc.store_scatter(ssel_v, [ranks[v]], sis[v], mask=m)
            for u in range((NQ + 15) // 16):
                pos = iota + (kk + 16 * u)
                val = iota + (ns + 16 * u)
                m = pos < nnew
                vc = jnp.minimum(val, np_in - 1)
                sq = plsc.load_gather(s_v, [vc], mask=m)
                plsc.store_scatter(idx_v, [pos], val + base, mask=m)
                plsc.store_scatter(ssel_v, [pos], sq, mask=m)
            pltpu.async_copy(x_hbm.at[idx_v], rows_v, sem).wait()
            pltpu.sync_copy(rows_v, rows_hbm.at[pl.ds(b * 128, 128)])
            pltpu.sync_copy(ssel_v, ssel_hbm.at[b])

    return sc_pool(x_flat, z)


def kernel(X, params):
    Xc, z0 = _tc_a(X, params)
    rows0, ssel0 = _sc_pool_call(Xc.reshape(B * 128, 128),
                                 z0.reshape(B, 128), 128, 128, 104)
    Xc2, z1 = _tc_mid(rows0.reshape(B, 128, 128), ssel0, params,
                      128, 104, 97, 'down_mlp_1', 'down_gcn_1', 'pool_1')
    rows1, ssel1 = _sc_pool_call(Xc2.reshape(B * 104, 128),
                                 z1.reshape(B, 128), 104, 97, 80)
    out = _tc_mid(rows1.reshape(B, 128, 128), ssel1, params,
                  128, 80, 75, 'bottom_mlp', 'bottom_gcn', None)
    return out[:, :75, :]


# speedup vs baseline: 1.0378x; 1.0378x over previous
import functools

import jax
import jax.numpy as jnp
from jax import lax
from jax.experimental import pallas as pl
from jax.experimental.pallas import tpu as pltpu
from jax.experimental.pallas import tpu_sc as plsc

B = 4
D = 96
NQ = 25
BN_C = float(1.0 / (1.0 + 1e-5) ** 0.5)


def _dot(a, b):
    return jnp.dot(a, b, preferred_element_type=jnp.float32)


def _mlp_attention(Xc, nv, w1, w2, w3, w4, w5, b5):
    np_, _ = Xc.shape
    lrelu = lambda v: jax.nn.leaky_relu(v, 0.01)
    G = 4 if np_ % 32 == 0 else 2
    g = np_ // G

    def chain(p):
        h = lrelu(_dot(p, w1))
        h = lrelu(_dot(h, w2))
        h = lrelu(_dot(h, w3))
        h = lrelu(_dot(h, w4))
        return _dot(h, w5) + b5

    blocks = {}
    for ci in range(G):
        xi = Xc[ci * g:(ci + 1) * g]
        for cj in range(ci, G):
            xj = Xc[cj * g:(cj + 1) * g]
            dif = jnp.abs(xi[:, None, :] - xj[None, :, :])
            lg = chain(dif.reshape(g * g, D)).reshape(g, g)
            blocks[(ci, cj)] = lg
    rows = []
    for ci in range(G):
        row = [blocks[(ci, cj)] if ci <= cj else blocks[(cj, ci)].T
               for cj in range(G)]
        rows.append(jnp.concatenate(row, axis=1))
    logits = jnp.concatenate(rows, axis=0)
    if nv < np_:
        jj = jax.lax.broadcasted_iota(jnp.int32, (np_, np_), 1)
        logits = jnp.where(jj < nv, logits, -1e30)
    return jax.nn.softmax(logits, axis=-1)


def _stage(Xc, nv, w1, w2, w3, w4, w5, b5, gw, gb):
    A = _mlp_attention(Xc, nv, w1, w2, w3, w4, w5, b5)
    return _dot(_dot(A, Xc), gw) + gb


def _score_row(Xc, wp, bp):
    return lax.dot_general(
        wp, Xc, (((0,), (1,)), ((), ())),
        preferred_element_type=jnp.float32) + bp


def _tc_a_kernel(x_ref, *refs):
    ws = [r[:] for r in refs[:18]]
    xc_out, z_out = refs[18], refs[19]
    Xc = x_ref[0]
    Xc = _stage(Xc, 128, *ws[0:8])
    Xc = _stage(Xc, 128, *ws[8:16])
    xc_out[0] = jnp.concatenate(
        [Xc, jnp.zeros((Xc.shape[0], 128 - D), jnp.float32)], axis=1)
    z_out[0] = _score_row(Xc, ws[16], ws[17])


def _tc_mid_kernel(np_use, nv, with_score, rows_ref, ssel_ref, *refs):
    nw = 10 if with_score else 8
    ws = [r[:] for r in refs[:nw]]
    Xc = (rows_ref[0] * ssel_ref[0])[:np_use, :D]
    Xc = _stage(Xc, nv, *ws[:8])
    if with_score:
        refs[nw][0] = jnp.concatenate(
            [Xc, jnp.zeros((np_use, 128 - D), jnp.float32)], axis=1)
        z = _score_row(Xc, ws[8], ws[9])
        if np_use < 128:
            z = jnp.concatenate(
                [z, jnp.zeros((1, 128 - np_use), jnp.float32)], axis=1)
        refs[nw + 1][0] = z
    else:
        refs[nw][0] = Xc


def _mlp_args(params, name):
    mp = params[name]
    return [mp['w1'] * BN_C, mp['w2'] * BN_C, mp['w3'] * BN_C,
            mp['w4'] * BN_C, mp['w5'], mp['b5'].reshape(1, 1)]


def _gcn_args(params, name):
    gp = params[name]
    return [gp['w1'], gp['b1'].reshape(1, D)]


def _pool_args(params, name):
    pp = params[name]
    return [pp['wp'], pp['bp'].reshape(1, 1)]


def _wspecs(args):
    return [pl.BlockSpec(a.shape, lambda b, _n=a.ndim: (0,) * _n)
            for a in args]


def _tc_a(X, params):
    args = (_mlp_args(params, 'start_mlp') + _gcn_args(params, 'start_gcn')
            + _mlp_args(params, 'down_mlp_0') + _gcn_args(params, 'down_gcn_0')
            + _pool_args(params, 'pool_0'))
    return pl.pallas_call(
        _tc_a_kernel,
        grid=(B,),
        in_specs=[pl.BlockSpec((1, 128, D), lambda b: (b, 0, 0))]
        + _wspecs(args),
        out_specs=[pl.BlockSpec((1, 128, 128), lambda b: (b, 0, 0)),
                   pl.BlockSpec((1, 1, 128), lambda b: (b, 0, 0))],
        out_shape=[jax.ShapeDtypeStruct((B, 128, 128), jnp.float32),
                   jax.ShapeDtypeStruct((B, 1, 128), jnp.float32)],
    )(X, *args)


def _tc_mid(rows, ssel, params, npad, np_use, nv, mlp, gcn, pool):
    args = _mlp_args(params, mlp) + _gcn_args(params, gcn)
    if pool is not None:
        args += _pool_args(params, pool)
    oc = 128 if pool is not None else D
    out_specs = [pl.BlockSpec((1, np_use, oc), lambda b: (b, 0, 0))]
    out_shape = [jax.ShapeDtypeStruct((B, np_use, oc), jnp.float32)]
    if pool is not None:
        out_specs.append(pl.BlockSpec((1, 1, 128), lambda b: (b, 0, 0)))
        out_shape.append(jax.ShapeDtypeStruct((B, 1, 128), jnp.float32))
    res = pl.pallas_call(
        functools.partial(_tc_mid_kernel, np_use, nv, pool is not None),
        grid=(B,),
        in_specs=[pl.BlockSpec((1, npad, 128), lambda b: (b, 0, 0)),
                  pl.BlockSpec((1, npad, 1), lambda b: (b, 0, 0))]
        + _wspecs(args),
        out_specs=out_specs,
        out_shape=out_shape,
    )(rows, ssel.reshape(B, npad, 1), *args)
    return res if pool is not None else res[0]


def _sc_pool_call(x_flat, z, np_in, nv, np_out):
    ns = nv - NQ
    kk = int(0.7 * ns)
    nnew = kk + NQ
    niv = (ns + 15) // 16

    mesh = plsc.VectorSubcoreMesh(core_axis_name="c", subcore_axis_name="s",
                                  num_cores=2, num_subcores=16)

    @functools.partial(
        pl.kernel,
        out_type=(jax.ShapeDtypeStruct((B * 128, 128), jnp.float32),
                  jax.ShapeDtypeStruct((B, 128), jnp.float32)),
        mesh=mesh,
        compiler_params=pltpu.CompilerParams(needs_layout_passes=False),
        scratch_types=[
            pltpu.VMEM((128,), jnp.float32),
            pltpu.VMEM((128,), jnp.int32),
            pltpu.VMEM((128,), jnp.float32),
            pltpu.VMEM((128, 128), jnp.float32),
            pltpu.SemaphoreType.DMA,
        ],
    )
    def sc_pool(x_hbm, z_hbm, rows_hbm, ssel_hbm, s_v, idx_v, ssel_v,
                rows_v, sem):
        cid = lax.axis_index("c")
        sid = lax.axis_index("s")
        wid = sid * 2 + cid

        @pl.when(wid < B)
        def _body():
            b = wid
            base = b * np_in
            pltpu.sync_copy(z_hbm.at[b], s_v)
            iota = lax.iota(jnp.int32, 16)
            for v in range(8):
                zz = s_v[pl.ds(16 * v, 16)]
                s_v[pl.ds(16 * v, 16)] = 1.0 / (1.0 + jnp.exp(-zz * 0.01))
            for v in range(8):
                idx_v[pl.ds(16 * v, 16)] = jnp.zeros((16,), jnp.int32) + base
                ssel_v[pl.ds(16 * v, 16)] = jnp.zeros((16,), jnp.float32)
            sis = [s_v[pl.ds(16 * v, 16)] for v in range(niv)]
            iis = [iota + 16 * v for v in range(niv)]

            def jbody(j, ranks):
                sj = plsc.load_gather(s_v, [jnp.zeros((16,), jnp.int32) + j])
                out = []
                for v in range(niv):
                    cond = ((sj > sis[v])
                            | ((sj == sis[v]) & (j < iis[v])))
                    out.append(ranks[v] + cond.astype(jnp.int32))
                return tuple(out)

            ranks = lax.fori_loop(
                0, ns, jbody,
                tuple(jnp.zeros((16,), jnp.int32) for _ in range(niv)))
            for v in range(niv):
                m = (ranks[v] < kk) & (iis[v] < ns)
                plsc.store_scatter(idx_v, [ranks[v]], iis[v] + base, mask=m)
                plsc.store_scatter(ssel_v, [ranks[v]], sis[v], mask=m)
            for u in range((NQ + 15) // 16):
                pos = iota + (kk + 16 * u)
                val = iota + (ns + 16 * u)
                m = pos < nnew
                vc = jnp.minimum(val, np_in - 1)
                sq = plsc.load_gather(s_v, [vc], mask=m)
                plsc.store_scatter(idx_v, [pos], val + base, mask=m)
                plsc.store_scatter(ssel_v, [pos], sq, mask=m)
            pltpu.async_copy(x_hbm.at[idx_v], rows_v, sem).wait()
            pltpu.sync_copy(rows_v, rows_hbm.at[pl.ds(b * 128, 128)])
            pltpu.sync_copy(ssel_v, ssel_hbm.at[b])

    return sc_pool(x_flat, z)


def kernel(X, params):
    Xc, z0 = _tc_a(X, params)
    rows0, ssel0 = _sc_pool_call(Xc.reshape(B * 128, 128),
                                 z0.reshape(B, 128), 128, 128, 104)
    Xc2, z1 = _tc_mid(rows0.reshape(B, 128, 128), ssel0, params,
                      128, 104, 97, 'down_mlp_1', 'down_gcn_1', 'pool_1')
    rows1, ssel1 = _sc_pool_call(Xc2.reshape(B * 104, 128),
                                 z1.reshape(B, 128), 104, 97, 80)
    out = _tc_mid(rows1.reshape(B, 128, 128), ssel1, params,
                  128, 80, 75, 'bottom_mlp', 'bottom_gcn', None)
    return out[:, :75, :]
